# baseline (speedup 1.0000x reference)
import jax
import jax.numpy as jnp
from jax.experimental import pallas as pl
from jax.experimental.pallas import tpu as pltpu


def _moe_body(x_ref, rw_ref, bu_ref, bd_ref, wup_hbm, wdn_hbm, out_ref,
              gates_ref, tbl_ref, ubuf, dbuf, sem_u, sem_d):
    x = x_ref[...]
    rw = rw_ref[...]
    B, E = x.shape[0], rw.shape[0]

    logits = jnp.dot(x, rw.T, preferred_element_type=jnp.float32)
    col = jax.lax.broadcasted_iota(jnp.int32, (B, E), 1)
    m1 = jnp.max(logits, axis=1, keepdims=True)
    i1 = jnp.min(jnp.where(logits == m1, col, E), axis=1, keepdims=True)
    masked = jnp.where(col == i1, -jnp.inf, logits)
    m2 = jnp.max(masked, axis=1, keepdims=True)
    i2 = jnp.min(jnp.where(masked == m2, col, E), axis=1, keepdims=True)
    t = jnp.exp(m2 - m1)
    w1 = 1.0 / (1.0 + t)
    w2 = t / (1.0 + t)
    sel1 = (col == i1)
    sel2 = (col == i2)
    gates_ref[...] = (w1 * sel1 + w2 * sel2).T

    act = jnp.max((sel1 | sel2).astype(jnp.float32), axis=0, keepdims=True)
    lt = (jax.lax.broadcasted_iota(jnp.int32, (E, E), 0) <
          jax.lax.broadcasted_iota(jnp.int32, (E, E), 1)).astype(jnp.float32)
    pos = jnp.dot(act, lt, preferred_element_type=jnp.float32)
    nactf = jnp.sum(act, dtype=jnp.float32)
    pos_c = pos.astype(jnp.int32).T
    act_c = act.astype(jnp.int32).T
    slot_m = jax.lax.broadcasted_iota(jnp.int32, (E, E), 1)
    e_m = jax.lax.broadcasted_iota(jnp.int32, (E, E), 0)
    take = (pos_c == slot_m) & (act_c == 1)
    order = jnp.sum(jnp.where(take, e_m, 0), axis=0, keepdims=True)
    ncol = jnp.full((E, 1), nactf, jnp.float32).astype(jnp.int32)
    tbl_ref[...] = jnp.concatenate(
        [order.T, ncol, jnp.zeros((E, 6), jnp.int32)], axis=1)

    nact = tbl_ref[0, 1]
    out_ref[...] = jnp.zeros_like(out_ref)
    U = ubuf.shape[1] * 4
    CH = U // 4
    DH = x.shape[1] // 2

    def issue(half, i):
        e = tbl_ref[i, 0]
        for c in range(4):
            pltpu.make_async_copy(wup_hbm.at[e, pl.ds(c * CH, CH), :],
                                  ubuf.at[half * 4 + c],
                                  sem_u.at[half * 4 + c]).start()
        for c in range(2):
            pltpu.make_async_copy(wdn_hbm.at[e, pl.ds(c * DH, DH), :],
                                  dbuf.at[half * 2 + c],
                                  sem_d.at[half * 2 + c]).start()

    issue(0, 0)

    def step(i, carry):
        half = jax.lax.rem(i, 2)

        @pl.when(i + 1 < nact)
        def _prefetch():
            issue(jax.lax.rem(i + 1, 2), i + 1)

        e = tbl_ref[i, 0]
        g = gates_ref[e, :].reshape(-1, 1)
        hs = []
        for c in range(4):
            pltpu.make_async_copy(wup_hbm.at[e, pl.ds(c * CH, CH), :],
                                  ubuf.at[half * 4 + c],
                                  sem_u.at[half * 4 + c]).wait()
            hs.append(jnp.dot(x, ubuf[half * 4 + c].T,
                              preferred_element_type=jnp.float32))
        h = jnp.concatenate(hs, axis=1)
        h = h + bu_ref[e, :]
        h = 0.5 * h * (1.0 + jax.lax.erf(h * 0.7071067811865476))
        h = h * g
        for c in range(2):
            pltpu.make_async_copy(wdn_hbm.at[e, pl.ds(c * DH, DH), :],
                                  dbuf.at[half * 2 + c],
                                  sem_d.at[half * 2 + c]).wait()
            out_ref[:, c * DH:(c + 1) * DH] += jnp.dot(
                h, dbuf[half * 2 + c].T, preferred_element_type=jnp.float32)
        out_ref[...] += g * bd_ref[e, :]
        return carry

    jax.lax.fori_loop(0, nact, step, 0)


def kernel(x, expert_weights_up, expert_weights_down, expert_biases_up,
           expert_biases_down, router_weight):
    B, S, D = x.shape
    E, U, _ = expert_weights_up.shape
    x2d = x.reshape(B * S, D)

    out = pl.pallas_call(
        _moe_body,
        in_specs=[
            pl.BlockSpec((B * S, D), lambda: (0, 0)),
            pl.BlockSpec((E, D), lambda: (0, 0)),
            pl.BlockSpec((E, U), lambda: (0, 0)),
            pl.BlockSpec((E, D), lambda: (0, 0)),
            pl.BlockSpec(memory_space=pltpu.MemorySpace.HBM),
            pl.BlockSpec(memory_space=pltpu.MemorySpace.HBM),
        ],
        out_specs=pl.BlockSpec((B * S, D), lambda: (0, 0)),
        out_shape=jax.ShapeDtypeStruct((B * S, D), jnp.float32),
        scratch_shapes=[
            pltpu.VMEM((E, B * S), jnp.float32),
            pltpu.VMEM((E, 8), jnp.int32),
            pltpu.VMEM((8, U // 4, D), jnp.float32),
            pltpu.VMEM((4, D // 2, U), jnp.float32),
            pltpu.SemaphoreType.DMA((8,)),
            pltpu.SemaphoreType.DMA((4,)),
        ],
    )(x2d, router_weight, expert_biases_up, expert_biases_down,
      expert_weights_up, expert_weights_down)
    return out.reshape(B, S, D)


# speedup vs baseline: 1.0148x; 1.0148x over previous
import jax
import jax.numpy as jnp
from jax.experimental import pallas as pl
from jax.experimental.pallas import tpu as pltpu


def _moe_body(x_ref, rw_ref, bu_ref, bd_ref, wup_hbm, wdn_hbm, out_ref,
              gates_ref, tbl_ref, ubuf, dbuf, sem_u, sem_d, sem_p):
    x = x_ref[...]
    rw = rw_ref[...]
    B, E = x.shape[0], rw.shape[0]
    D = x.shape[1]
    U = wup_hbm.shape[1]
    CH = U // 2
    QH = U // 4
    DH = D // 2

    logits = jnp.dot(x, rw.T, preferred_element_type=jnp.float32)
    col = jax.lax.broadcasted_iota(jnp.int32, (B, E), 1)
    m1 = jnp.max(logits, axis=1, keepdims=True)
    i1 = jnp.min(jnp.where(logits == m1, col, E), axis=1, keepdims=True)
    masked = jnp.where(col == i1, -jnp.inf, logits)
    m2 = jnp.max(masked, axis=1, keepdims=True)
    i2 = jnp.min(jnp.where(masked == m2, col, E), axis=1, keepdims=True)

    e0 = jnp.min(jnp.minimum(i1, i2))
    for q in range(4):
        pltpu.make_async_copy(wup_hbm.at[e0, pl.ds(q * QH, QH), :],
                              ubuf.at[q // 2, pl.ds((q % 2) * QH, QH), :],
                              sem_p.at[q]).start()
    for c in range(2):
        pltpu.make_async_copy(wdn_hbm.at[e0, pl.ds(c * DH, DH), :],
                              dbuf.at[c], sem_d.at[c]).start()

    t = jnp.exp(m2 - m1)
    w1 = 1.0 / (1.0 + t)
    w2 = t / (1.0 + t)
    sel1 = (col == i1)
    sel2 = (col == i2)
    gates_ref[...] = (w1 * sel1 + w2 * sel2).T

    act = jnp.max((sel1 | sel2).astype(jnp.float32), axis=0, keepdims=True)
    lt = (jax.lax.broadcasted_iota(jnp.int32, (E, E), 0) <
          jax.lax.broadcasted_iota(jnp.int32, (E, E), 1)).astype(jnp.float32)
    pos = jnp.dot(act, lt, preferred_element_type=jnp.float32)
    nactf = jnp.sum(act, dtype=jnp.float32)
    pos_c = pos.astype(jnp.int32).T
    act_c = act.astype(jnp.int32).T
    slot_m = jax.lax.broadcasted_iota(jnp.int32, (E, E), 1)
    e_m = jax.lax.broadcasted_iota(jnp.int32, (E, E), 0)
    take = (pos_c == slot_m) & (act_c == 1)
    order = jnp.sum(jnp.where(take, e_m, 0), axis=0, keepdims=True)
    ncol = jnp.full((E, 1), nactf, jnp.float32).astype(jnp.int32)
    tbl_ref[...] = jnp.concatenate(
        [order.T, ncol, jnp.zeros((E, 6), jnp.int32)], axis=1)

    nact = tbl_ref[0, 1]
    out_ref[...] = jnp.zeros_like(out_ref)

    def issue(half, i):
        e = tbl_ref[i, 0]
        for c in range(2):
            pltpu.make_async_copy(wup_hbm.at[e, pl.ds(c * CH, CH), :],
                                  ubuf.at[half * 2 + c],
                                  sem_u.at[half * 2 + c]).start()
        for c in range(2):
            pltpu.make_async_copy(wdn_hbm.at[e, pl.ds(c * DH, DH), :],
                                  dbuf.at[half * 2 + c],
                                  sem_d.at[half * 2 + c]).start()

    @pl.when(1 < nact)
    def _prefetch1():
        issue(1, 1)

    g0 = gates_ref[e0, :].reshape(-1, 1)
    hs0 = []
    for q in range(4):
        pltpu.make_async_copy(wup_hbm.at[e0, pl.ds(q * QH, QH), :],
                              ubuf.at[q // 2, pl.ds((q % 2) * QH, QH), :],
                              sem_p.at[q]).wait()
        hs0.append(jnp.dot(x, ubuf[q // 2, (q % 2) * QH:(q % 2) * QH + QH, :].T,
                           preferred_element_type=jnp.float32))
    h0 = jnp.concatenate(hs0, axis=1)
    h0 = h0 + bu_ref[e0, :]
    h0 = 0.5 * h0 * (1.0 + jax.lax.erf(h0 * 0.7071067811865476))
    h0 = h0 * g0
    for c in range(2):
        pltpu.make_async_copy(wdn_hbm.at[e0, pl.ds(c * DH, DH), :],
                              dbuf.at[c], sem_d.at[c]).wait()
        out_ref[:, c * DH:(c + 1) * DH] += jnp.dot(
            h0, dbuf[c].T, preferred_element_type=jnp.float32)
    out_ref[...] += g0 * bd_ref[e0, :]

    def step(i, carry):
        half = jax.lax.rem(i, 2)

        @pl.when(i + 1 < nact)
        def _prefetch():
            issue(jax.lax.rem(i + 1, 2), i + 1)

        e = tbl_ref[i, 0]
        g = gates_ref[e, :].reshape(-1, 1)
        hs = []
        for c in range(2):
            pltpu.make_async_copy(wup_hbm.at[e, pl.ds(c * CH, CH), :],
                                  ubuf.at[half * 2 + c],
                                  sem_u.at[half * 2 + c]).wait()
            hs.append(jnp.dot(x, ubuf[half * 2 + c].T,
                              preferred_element_type=jnp.float32))
        h = jnp.concatenate(hs, axis=1)
        h = h + bu_ref[e, :]
        h = 0.5 * h * (1.0 + jax.lax.erf(h * 0.7071067811865476))
        h = h * g
        for c in range(2):
            pltpu.make_async_copy(wdn_hbm.at[e, pl.ds(c * DH, DH), :],
                                  dbuf.at[half * 2 + c],
                                  sem_d.at[half * 2 + c]).wait()
            out_ref[:, c * DH:(c + 1) * DH] += jnp.dot(
                h, dbuf[half * 2 + c].T, preferred_element_type=jnp.float32)
        out_ref[...] += g * bd_ref[e, :]
        return carry

    jax.lax.fori_loop(1, nact, step, 0)


def kernel(x, expert_weights_up, expert_weights_down, expert_biases_up,
           expert_biases_down, router_weight):
    B, S, D = x.shape
    E, U, _ = expert_weights_up.shape
    x2d = x.reshape(B * S, D)

    out = pl.pallas_call(
        _moe_body,
        in_specs=[
            pl.BlockSpec((B * S, D), lambda: (0, 0)),
            pl.BlockSpec((E, D), lambda: (0, 0)),
            pl.BlockSpec((E, U), lambda: (0, 0)),
            pl.BlockSpec((E, D), lambda: (0, 0)),
            pl.BlockSpec(memory_space=pltpu.MemorySpace.HBM),
            pl.BlockSpec(memory_space=pltpu.MemorySpace.HBM),
        ],
        out_specs=pl.BlockSpec((B * S, D), lambda: (0, 0)),
        out_shape=jax.ShapeDtypeStruct((B * S, D), jnp.float32),
        scratch_shapes=[
            pltpu.VMEM((E, B * S), jnp.float32),
            pltpu.VMEM((E, 8), jnp.int32),
            pltpu.VMEM((4, U // 2, D), jnp.float32),
            pltpu.VMEM((4, D // 2, U), jnp.float32),
            pltpu.SemaphoreType.DMA((4,)),
            pltpu.SemaphoreType.DMA((4,)),
            pltpu.SemaphoreType.DMA((4,)),
        ],
    )(x2d, router_weight, expert_biases_up, expert_biases_down,
      expert_weights_up, expert_weights_down)
    return out.reshape(B, S, D)
